# baseline (speedup 1.0000x reference)
import functools

import jax
import jax.numpy as jnp
from jax import lax
from jax.experimental import pallas as pl
from jax.experimental.pallas import tpu as pltpu
from jax.experimental.pallas import tpu_sc as plsc

N = 10000
E = 320000
F = 128
K = 16
NC = 2
NS = 16
NW = NC * NS
CB = 128
NCH = 80
EW = NCH * CB
E_PAD = NW * EW
N_ACC = N + 128
NBUF = 4
ZC = 80
NZCH = N // ZC


def _mesh():
    return plsc.VectorSubcoreMesh(core_axis_name="c", subcore_axis_name="s",
                                  num_cores=NC, num_subcores=NS)


def _zero_fill(zb, width):
    def body(i, carry):
        for k8 in range(width // 16):
            zb[i, pl.ds(k8 * 16, 16)] = jnp.zeros((16,), jnp.float32)
        return carry
    lax.fori_loop(0, ZC, body, 0)


def _zero_acc(zb, acc, sid):
    def body(k, carry):
        c = sid + k * NS

        @pl.when(c < NZCH)
        def _():
            pltpu.sync_copy(zb, acc.at[pl.ds(c * ZC, ZC)])
        return carry
    lax.fori_loop(0, (NZCH + NS - 1) // NS, body, 0)


def _write_out(acc, out_hbm, cid, width, h, sid):
    def body(k, carry):
        c = sid + k * NS

        @pl.when(c < NZCH)
        def _():
            pltpu.sync_copy(acc.at[pl.ds(c * ZC, ZC)],
                            out_hbm.at[cid, pl.ds(c * ZC, ZC),
                                       pl.ds(h * width, width)])
        return carry
    lax.fori_loop(0, (NZCH + NS - 1) // NS, body, 0)


def _sc_deg_body(dst_hbm, out_hbm, idx_v, ones_v, zb_v, acc):
    cid = lax.axis_index("c")
    sid = lax.axis_index("s")
    wid = sid * NC + cid

    def fill_ones(i, carry):
        ones_v[i, :] = jnp.full((16,), 1.0, jnp.float32)
        return carry
    lax.fori_loop(0, CB, fill_ones, 0)
    _zero_fill(zb_v, K)
    _zero_acc(zb_v, acc, sid)
    plsc.subcore_barrier()

    pltpu.sync_copy(dst_hbm.at[wid], idx_v)

    def body(j, carry):
        pltpu.sync_copy(ones_v, acc.at[idx_v.at[j]], add=True)
        return carry
    lax.fori_loop(0, NCH, body, 0)
    plsc.subcore_barrier()

    _write_out(acc, out_hbm, cid, K, 0, sid)


def _sc_edge_body(width, nh, *refs):
    rows_list = refs[:nh]
    src_hbm, dst_hbm, out_hbm = refs[nh:nh + 3]
    idxs_v, idxd_v = refs[nh + 3:nh + 5]
    bufs = refs[nh + 5:nh + 5 + NBUF]
    zb_v, acc = refs[nh + 5 + NBUF:nh + 7 + NBUF]
    sems = refs[nh + 7 + NBUF:nh + 7 + 2 * NBUF]
    cid = lax.axis_index("c")
    sid = lax.axis_index("s")
    wid = sid * NC + cid

    _zero_fill(zb_v, width)
    _zero_acc(zb_v, acc, sid)
    plsc.subcore_barrier()

    pltpu.sync_copy(src_hbm.at[wid], idxs_v)
    pltpu.sync_copy(dst_hbm.at[wid], idxd_v)

    for h in range(nh):
        rows_hbm = rows_list[h]
        for b in range(NBUF):
            pltpu.async_copy(rows_hbm.at[idxs_v.at[b]], bufs[b], sems[b])

        def outer(o, carry):
            j0 = o * NBUF
            for b in range(NBUF):
                j = j0 + b
                pltpu.make_async_copy(rows_hbm.at[idxs_v.at[j]],
                                      bufs[b], sems[b]).wait()
                pltpu.sync_copy(bufs[b], acc.at[idxd_v.at[j]], add=True)

                @pl.when(j + NBUF < NCH)
                def _():
                    pltpu.async_copy(rows_hbm.at[idxs_v.at[j + NBUF]],
                                     bufs[b], sems[b])
            return carry
        lax.fori_loop(0, NCH // NBUF, outer, 0)
        plsc.subcore_barrier()

        _write_out(acc, out_hbm, cid, width, h, sid)
        if h + 1 < nh:
            _zero_acc(zb_v, acc, sid)
            plsc.subcore_barrier()


NCH2 = E_PAD // (NS * CB)


def _sc_edge_split_body(rows_hbm, src_hbm, dst_hbm, out_hbm,
                        idxs_v, idxd_v, b0, b1, b2, b3, zb_v, acc,
                        s0, s1, s2, s3):
    bufs = (b0, b1, b2, b3)
    sems = (s0, s1, s2, s3)
    cid = lax.axis_index("c")
    sid = lax.axis_index("s")

    _zero_fill(zb_v, F // 2)
    _zero_acc(zb_v, acc, sid)

    pltpu.sync_copy(src_hbm.at[sid], idxs_v)
    pltpu.sync_copy(dst_hbm.at[sid], idxd_v)

    def xform(i, carry):
        for k8 in range(CB // 16):
            v = idxs_v[i, pl.ds(k8 * 16, 16)]
            idxs_v[i, pl.ds(k8 * 16, 16)] = v * 2 + cid
        return carry
    lax.fori_loop(0, NCH2, xform, 0)
    plsc.subcore_barrier()

    for b in range(NBUF):
        pltpu.async_copy(rows_hbm.at[idxs_v.at[b]], bufs[b], sems[b])

    def outer(o, carry):
        j0 = o * NBUF
        for b in range(NBUF):
            j = j0 + b
            pltpu.make_async_copy(rows_hbm.at[idxs_v.at[j]],
                                  bufs[b], sems[b]).wait()
            pltpu.sync_copy(bufs[b], acc.at[idxd_v.at[j]], add=True)

            @pl.when(j + NBUF < NCH2)
            def _():
                pltpu.async_copy(rows_hbm.at[idxs_v.at[j + NBUF]],
                                 bufs[b], sems[b])
        return carry
    lax.fori_loop(0, NCH2 // NBUF, outer, 0)
    plsc.subcore_barrier()

    def wout(k, carry):
        c = sid + k * NS

        @pl.when(c < NZCH)
        def _():
            pltpu.sync_copy(acc.at[pl.ds(c * ZC, ZC)],
                            out_hbm.at[pl.ds(c * ZC, ZC), cid])
        return carry
    lax.fori_loop(0, (NZCH + NS - 1) // NS, wout, 0)


def _make_edge_split_kernel():
    return pl.kernel(
        _sc_edge_split_body,
        out_type=jax.ShapeDtypeStruct((N, NC, F // 2), jnp.float32),
        mesh=_mesh(),
        compiler_params=pltpu.CompilerParams(use_tc_tiling_on_sc=False),
        scratch_types=(
            [pltpu.VMEM((NCH2, CB), jnp.int32),
             pltpu.VMEM((NCH2, CB), jnp.int32)]
            + [pltpu.VMEM((CB, F // 2), jnp.float32) for _ in range(NBUF)]
            + [pltpu.VMEM((ZC, F // 2), jnp.float32),
               pltpu.VMEM_SHARED((N_ACC, F // 2), jnp.float32)]
            + [pltpu.SemaphoreType.DMA for _ in range(NBUF)]
        ),
    )


def _make_deg_kernel():
    return pl.kernel(
        _sc_deg_body,
        out_type=jax.ShapeDtypeStruct((NC, N, F), jnp.float32),
        mesh=_mesh(),
        compiler_params=pltpu.CompilerParams(use_tc_tiling_on_sc=False),
        scratch_types=[
            pltpu.VMEM((NCH, CB), jnp.int32),
            pltpu.VMEM((CB, K), jnp.float32),
            pltpu.VMEM((ZC, K), jnp.float32),
            pltpu.VMEM_SHARED((N_ACC, K), jnp.float32),
        ],
    )


def _make_edge_kernel(width, nh):
    return pl.kernel(
        functools.partial(_sc_edge_body, width, nh),
        out_type=jax.ShapeDtypeStruct((NC, N, F), jnp.float32),
        mesh=_mesh(),
        compiler_params=pltpu.CompilerParams(use_tc_tiling_on_sc=False),
        scratch_types=(
            [pltpu.VMEM((NCH, CB), jnp.int32),
             pltpu.VMEM((NCH, CB), jnp.int32)]
            + [pltpu.VMEM((CB, width), jnp.float32) for _ in range(NBUF)]
            + [pltpu.VMEM((ZC, width), jnp.float32),
               pltpu.VMEM_SHARED((N_ACC, width), jnp.float32)]
            + [pltpu.SemaphoreType.DMA for _ in range(NBUF)]
        ),
    )


_BR = 1000


def _tc1a_body(x_ref, w1_ref, h1_ref):
    h1_ref[...] = jnp.dot(x_ref[...], w1_ref[...],
                          preferred_element_type=jnp.float32)


def _tc1b_body(h1_ref, degp_ref, h1p_ref, dinv_ref):
    deg = degp_ref[0, :, 0:1] + degp_ref[1, :, 0:1] + 1.0
    dinv = lax.rsqrt(deg)
    h1p_ref[...] = dinv * h1_ref[...]
    dinv_ref[...] = jnp.broadcast_to(dinv, (_BR, F))


def _tc2a_body(aggb_ref, h1p_ref, dinv_ref, b1_ref, w2_ref,
               h_ref, h2p_ref):
    dinv = dinv_ref[:, 0:1]
    agg = aggb_ref[...] + h1p_ref[...]
    h = jnp.maximum(dinv * agg + b1_ref[...], 0.0)
    h_ref[...] = h
    h2 = jnp.dot(h, w2_ref[...], preferred_element_type=jnp.float32)
    h2p_ref[...] = dinv * h2


def _tc2b_body(h_ref, ct_ref, csq_ref, q_ref):
    h = h_ref[...]
    hsq = jnp.sum(h * h, axis=1, keepdims=True)
    cross = jnp.dot(h, ct_ref[...], preferred_element_type=jnp.float32)
    dist = hsq - 2.0 * cross + csq_ref[...]
    qun = 1.0 / (1.0 + dist)
    q_ref[...] = qun / jnp.sum(qun, axis=1, keepdims=True)


def _tc3_body(aggc_ref, h2p_ref, dinv_ref, b2_ref, out_ref):
    aggc = aggc_ref[0, :, :K] + aggc_ref[1, :, :K]
    logits = dinv_ref[:, 0:1] * (aggc + h2p_ref[...])
    logits = logits + b2_ref[...]
    m = jnp.max(logits, axis=1, keepdims=True)
    s = logits - m
    out_ref[...] = s - jnp.log(jnp.sum(jnp.exp(s), axis=1, keepdims=True))


def _row_spec(width):
    return pl.BlockSpec((_BR, width), lambda i: (i, 0))


def _full_spec(shape):
    return pl.BlockSpec(shape, lambda i: tuple(0 for _ in shape))


def _part_spec(width):
    return pl.BlockSpec((NC, _BR, width), lambda i: (0, i, 0))


_GRID = N // _BR


def kernel(x, edge_index, W1, b1, W2, b2, cluster_centers):
    ei = edge_index.astype(jnp.int32)
    pad_ids = jnp.arange(E_PAD - E, dtype=jnp.int32)
    fill = jnp.stack([pad_ids % N, N + (pad_ids % 128)])
    ei = jnp.concatenate([ei, fill], axis=1)
    src3 = ei[0].reshape(NW, NCH, CB)
    dst3 = ei[1].reshape(NW, NCH, CB)
    src2 = ei[0].reshape(NS, NCH2, CB)
    dst2 = ei[1].reshape(NS, NCH2, CB)

    degp = _make_deg_kernel()(dst3)

    h1 = pl.pallas_call(
        _tc1a_body,
        grid=(_GRID,),
        in_specs=[_row_spec(F), _full_spec((F, F))],
        out_specs=_row_spec(F),
        out_shape=jax.ShapeDtypeStruct((N, F), jnp.float32),
    )(x, W1)

    h1p, dinv = pl.pallas_call(
        _tc1b_body,
        grid=(_GRID,),
        in_specs=[_row_spec(F), _part_spec(F)],
        out_specs=[_row_spec(F), _row_spec(F)],
        out_shape=[jax.ShapeDtypeStruct((N, F), jnp.float32),
                   jax.ShapeDtypeStruct((N, F), jnp.float32)],
    )(h1, degp)

    aggb = _make_edge_split_kernel()(
        h1p.reshape(2 * N, F // 2), src2, dst2).reshape(N, F)

    b1r = b1.reshape(1, F)
    b2r = b2.reshape(1, K)
    ct = cluster_centers.T
    csq = jnp.sum(cluster_centers * cluster_centers, axis=1).reshape(1, K)

    h, h2p = pl.pallas_call(
        _tc2a_body,
        grid=(_GRID,),
        in_specs=[_row_spec(F), _row_spec(F), _row_spec(F),
                  _full_spec((1, F)), _full_spec((F, K))],
        out_specs=[_row_spec(F), _row_spec(K)],
        out_shape=[jax.ShapeDtypeStruct((N, F), jnp.float32),
                   jax.ShapeDtypeStruct((N, K), jnp.float32)],
    )(aggb, h1p, dinv, b1r, W2)

    aggc = _make_edge_kernel(K, 1)(h2p, src3, dst3)

    q = pl.pallas_call(
        _tc2b_body,
        grid=(_GRID,),
        in_specs=[_row_spec(F), _full_spec((F, K)), _full_spec((1, K))],
        out_specs=_row_spec(K),
        out_shape=jax.ShapeDtypeStruct((N, K), jnp.float32),
    )(h, ct, csq)

    logsm = pl.pallas_call(
        _tc3_body,
        grid=(_GRID,),
        in_specs=[_part_spec(F), _row_spec(K), _row_spec(F),
                  _full_spec((1, K))],
        out_specs=_row_spec(K),
        out_shape=jax.ShapeDtypeStruct((N, K), jnp.float32),
    )(aggc, h2p, dinv, b2r)

    return (logsm, q)


# speedup vs baseline: 1.1755x; 1.1755x over previous
import functools

import jax
import jax.numpy as jnp
from jax import lax
from jax.experimental import pallas as pl
from jax.experimental.pallas import tpu as pltpu
from jax.experimental.pallas import tpu_sc as plsc

N = 10000
E = 320000
F = 128
K = 16
NC = 2
NS = 16
NW = NC * NS
CB = 128
NCH = 80
EW = NCH * CB
E_PAD = NW * EW
N_ACC = N + 128
NBUF = 4
ZC = 80
NZCH = N // ZC


def _mesh():
    return plsc.VectorSubcoreMesh(core_axis_name="c", subcore_axis_name="s",
                                  num_cores=NC, num_subcores=NS)


def _zero_fill(zb, width):
    def body(i, carry):
        for k8 in range(width // 16):
            zb[i, pl.ds(k8 * 16, 16)] = jnp.zeros((16,), jnp.float32)
        return carry
    lax.fori_loop(0, ZC, body, 0)


def _zero_acc(zb, acc, sid):
    def body(k, carry):
        c = sid + k * NS

        @pl.when(c < NZCH)
        def _():
            pltpu.sync_copy(zb, acc.at[pl.ds(c * ZC, ZC)])
        return carry
    lax.fori_loop(0, (NZCH + NS - 1) // NS, body, 0)


def _write_out(acc, out_hbm, cid, width, h, sid):
    def body(k, carry):
        c = sid + k * NS

        @pl.when(c < NZCH)
        def _():
            pltpu.sync_copy(acc.at[pl.ds(c * ZC, ZC)],
                            out_hbm.at[cid, pl.ds(c * ZC, ZC),
                                       pl.ds(h * width, width)])
        return carry
    lax.fori_loop(0, (NZCH + NS - 1) // NS, body, 0)


def _sc_deg_body(dst_hbm, out_hbm, idx_v, ones_v, zb_v, acc):
    cid = lax.axis_index("c")
    sid = lax.axis_index("s")
    wid = sid * NC + cid

    def fill_ones(i, carry):
        ones_v[i, :] = jnp.full((16,), 1.0, jnp.float32)
        return carry
    lax.fori_loop(0, CB, fill_ones, 0)
    _zero_fill(zb_v, K)
    _zero_acc(zb_v, acc, sid)
    plsc.subcore_barrier()

    pltpu.sync_copy(dst_hbm.at[wid], idx_v)

    def body(j, carry):
        pltpu.sync_copy(ones_v, acc.at[idx_v.at[j]], add=True)
        return carry
    lax.fori_loop(0, NCH, body, 0)
    plsc.subcore_barrier()

    _write_out(acc, out_hbm, cid, K, 0, sid)


def _sc_edge_body(width, nh, *refs):
    rows_list = refs[:nh]
    src_hbm, dst_hbm, out_hbm = refs[nh:nh + 3]
    idxs_v, idxd_v = refs[nh + 3:nh + 5]
    bufs = refs[nh + 5:nh + 5 + NBUF]
    zb_v, acc = refs[nh + 5 + NBUF:nh + 7 + NBUF]
    sems = refs[nh + 7 + NBUF:nh + 7 + 2 * NBUF]
    cid = lax.axis_index("c")
    sid = lax.axis_index("s")
    wid = sid * NC + cid

    _zero_fill(zb_v, width)
    _zero_acc(zb_v, acc, sid)
    plsc.subcore_barrier()

    pltpu.sync_copy(src_hbm.at[wid], idxs_v)
    pltpu.sync_copy(dst_hbm.at[wid], idxd_v)

    for h in range(nh):
        rows_hbm = rows_list[h]
        for b in range(NBUF):
            pltpu.async_copy(rows_hbm.at[idxs_v.at[b]], bufs[b], sems[b])

        def outer(o, carry):
            j0 = o * NBUF
            for b in range(NBUF):
                j = j0 + b
                pltpu.make_async_copy(rows_hbm.at[idxs_v.at[j]],
                                      bufs[b], sems[b]).wait()
                pltpu.sync_copy(bufs[b], acc.at[idxd_v.at[j]], add=True)

                @pl.when(j + NBUF < NCH)
                def _():
                    pltpu.async_copy(rows_hbm.at[idxs_v.at[j + NBUF]],
                                     bufs[b], sems[b])
            return carry
        lax.fori_loop(0, NCH // NBUF, outer, 0)
        plsc.subcore_barrier()

        _write_out(acc, out_hbm, cid, width, h, sid)
        if h + 1 < nh:
            _zero_acc(zb_v, acc, sid)
            plsc.subcore_barrier()


NCH2 = E_PAD // (NS * CB)


def _sc_edge_split_body(rows_hbm, src_hbm, dst_hbm, out_hbm,
                        idxs_v, idxd_v, b0, b1, b2, b3, zb_v, acc,
                        s0, s1, s2, s3):
    bufs = (b0, b1, b2, b3)
    sems = (s0, s1, s2, s3)
    cid = lax.axis_index("c")
    sid = lax.axis_index("s")

    _zero_fill(zb_v, F // 2)
    _zero_acc(zb_v, acc, sid)

    pltpu.sync_copy(src_hbm.at[sid], idxs_v)
    pltpu.sync_copy(dst_hbm.at[sid], idxd_v)

    def xform(i, carry):
        for k8 in range(CB // 16):
            v = idxs_v[i, pl.ds(k8 * 16, 16)]
            idxs_v[i, pl.ds(k8 * 16, 16)] = v * 2 + cid
        return carry
    lax.fori_loop(0, NCH2, xform, 0)
    plsc.subcore_barrier()

    for b in range(NBUF):
        pltpu.async_copy(rows_hbm.at[idxs_v.at[b]], bufs[b], sems[b])

    def outer(o, carry):
        j0 = o * NBUF
        for b in range(NBUF):
            j = j0 + b
            pltpu.make_async_copy(rows_hbm.at[idxs_v.at[j]],
                                  bufs[b], sems[b]).wait()
            pltpu.sync_copy(bufs[b], acc.at[idxd_v.at[j]], add=True)

            @pl.when(j + NBUF < NCH2)
            def _():
                pltpu.async_copy(rows_hbm.at[idxs_v.at[j + NBUF]],
                                 bufs[b], sems[b])
        return carry
    lax.fori_loop(0, NCH2 // NBUF, outer, 0)
    plsc.subcore_barrier()

    def wout(k, carry):
        c = sid + k * NS

        @pl.when(c < NZCH)
        def _():
            pltpu.sync_copy(acc.at[pl.ds(c * ZC, ZC)],
                            out_hbm.at[pl.ds(c * ZC, ZC),
                                       pl.ds(cid * (F // 2), F // 2)])
        return carry
    lax.fori_loop(0, (NZCH + NS - 1) // NS, wout, 0)


def _make_edge_split_kernel():
    return pl.kernel(
        _sc_edge_split_body,
        out_type=jax.ShapeDtypeStruct((N, F), jnp.float32),
        mesh=_mesh(),
        compiler_params=pltpu.CompilerParams(use_tc_tiling_on_sc=False),
        scratch_types=(
            [pltpu.VMEM((NCH2, CB), jnp.int32),
             pltpu.VMEM((NCH2, CB), jnp.int32)]
            + [pltpu.VMEM((CB, F // 2), jnp.float32) for _ in range(NBUF)]
            + [pltpu.VMEM((ZC, F // 2), jnp.float32),
               pltpu.VMEM_SHARED((N_ACC, F // 2), jnp.float32)]
            + [pltpu.SemaphoreType.DMA for _ in range(NBUF)]
        ),
    )


def _make_deg_kernel():
    return pl.kernel(
        _sc_deg_body,
        out_type=jax.ShapeDtypeStruct((NC, N, F), jnp.float32),
        mesh=_mesh(),
        compiler_params=pltpu.CompilerParams(use_tc_tiling_on_sc=False),
        scratch_types=[
            pltpu.VMEM((NCH, CB), jnp.int32),
            pltpu.VMEM((CB, K), jnp.float32),
            pltpu.VMEM((ZC, K), jnp.float32),
            pltpu.VMEM_SHARED((N_ACC, K), jnp.float32),
        ],
    )


def _make_edge_kernel(width, nh):
    return pl.kernel(
        functools.partial(_sc_edge_body, width, nh),
        out_type=jax.ShapeDtypeStruct((NC, N, F), jnp.float32),
        mesh=_mesh(),
        compiler_params=pltpu.CompilerParams(use_tc_tiling_on_sc=False),
        scratch_types=(
            [pltpu.VMEM((NCH, CB), jnp.int32),
             pltpu.VMEM((NCH, CB), jnp.int32)]
            + [pltpu.VMEM((CB, width), jnp.float32) for _ in range(NBUF)]
            + [pltpu.VMEM((ZC, width), jnp.float32),
               pltpu.VMEM_SHARED((N_ACC, width), jnp.float32)]
            + [pltpu.SemaphoreType.DMA for _ in range(NBUF)]
        ),
    )


_BR = 1000


def _tc1a_body(x_ref, w1_ref, h1_ref):
    h1_ref[...] = jnp.dot(x_ref[...], w1_ref[...],
                          preferred_element_type=jnp.float32)


def _tc1b_body(h1_ref, degp_ref, h1p_ref, dinv_ref):
    deg = degp_ref[0, :, 0:1] + degp_ref[1, :, 0:1] + 1.0
    dinv = lax.rsqrt(deg)
    h1p_ref[...] = dinv * h1_ref[...]
    dinv_ref[...] = jnp.broadcast_to(dinv, (_BR, F))


def _tc2a_body(aggb_ref, h1p_ref, dinv_ref, b1_ref, w2_ref,
               h_ref, h2p_ref):
    dinv = dinv_ref[:, 0:1]
    agg = aggb_ref[...] + h1p_ref[...]
    h = jnp.maximum(dinv * agg + b1_ref[...], 0.0)
    h_ref[...] = h
    h2 = jnp.dot(h, w2_ref[...], preferred_element_type=jnp.float32)
    h2p_ref[...] = dinv * h2


def _tc2b_body(h_ref, ct_ref, csq_ref, q_ref):
    h = h_ref[...]
    hsq = jnp.sum(h * h, axis=1, keepdims=True)
    cross = jnp.dot(h, ct_ref[...], preferred_element_type=jnp.float32)
    dist = hsq - 2.0 * cross + csq_ref[...]
    qun = 1.0 / (1.0 + dist)
    q_ref[...] = qun / jnp.sum(qun, axis=1, keepdims=True)


def _tc3_body(aggc_ref, h2p_ref, dinv_ref, b2_ref, out_ref):
    aggc = aggc_ref[0, :, :K] + aggc_ref[1, :, :K]
    logits = dinv_ref[:, 0:1] * (aggc + h2p_ref[...])
    logits = logits + b2_ref[...]
    m = jnp.max(logits, axis=1, keepdims=True)
    s = logits - m
    out_ref[...] = s - jnp.log(jnp.sum(jnp.exp(s), axis=1, keepdims=True))


def _row_spec(width):
    return pl.BlockSpec((_BR, width), lambda i: (i, 0))


def _full_spec(shape):
    return pl.BlockSpec(shape, lambda i: tuple(0 for _ in shape))


def _part_spec(width):
    return pl.BlockSpec((NC, _BR, width), lambda i: (0, i, 0))


_GRID = N // _BR


def kernel(x, edge_index, W1, b1, W2, b2, cluster_centers):
    ei = edge_index.astype(jnp.int32)
    pad_ids = jnp.arange(E_PAD - E, dtype=jnp.int32)
    fill = jnp.stack([pad_ids % N, N + (pad_ids % 128)])
    ei = jnp.concatenate([ei, fill], axis=1)
    src3 = ei[0].reshape(NW, NCH, CB)
    dst3 = ei[1].reshape(NW, NCH, CB)
    src2 = ei[0].reshape(NS, NCH2, CB)
    dst2 = ei[1].reshape(NS, NCH2, CB)

    degp = _make_deg_kernel()(dst3)

    h1 = pl.pallas_call(
        _tc1a_body,
        grid=(_GRID,),
        in_specs=[_row_spec(F), _full_spec((F, F))],
        out_specs=_row_spec(F),
        out_shape=jax.ShapeDtypeStruct((N, F), jnp.float32),
    )(x, W1)

    h1p, dinv = pl.pallas_call(
        _tc1b_body,
        grid=(_GRID,),
        in_specs=[_row_spec(F), _part_spec(F)],
        out_specs=[_row_spec(F), _row_spec(F)],
        out_shape=[jax.ShapeDtypeStruct((N, F), jnp.float32),
                   jax.ShapeDtypeStruct((N, F), jnp.float32)],
    )(h1, degp)

    aggb = _make_edge_split_kernel()(
        h1p.reshape(2 * N, F // 2), src2, dst2)

    b1r = b1.reshape(1, F)
    b2r = b2.reshape(1, K)
    ct = cluster_centers.T
    csq = jnp.sum(cluster_centers * cluster_centers, axis=1).reshape(1, K)

    h, h2p = pl.pallas_call(
        _tc2a_body,
        grid=(_GRID,),
        in_specs=[_row_spec(F), _row_spec(F), _row_spec(F),
                  _full_spec((1, F)), _full_spec((F, K))],
        out_specs=[_row_spec(F), _row_spec(K)],
        out_shape=[jax.ShapeDtypeStruct((N, F), jnp.float32),
                   jax.ShapeDtypeStruct((N, K), jnp.float32)],
    )(aggb, h1p, dinv, b1r, W2)

    aggc = _make_edge_kernel(K, 1)(h2p, src3, dst3)

    q = pl.pallas_call(
        _tc2b_body,
        grid=(_GRID,),
        in_specs=[_row_spec(F), _full_spec((F, K)), _full_spec((1, K))],
        out_specs=_row_spec(K),
        out_shape=jax.ShapeDtypeStruct((N, K), jnp.float32),
    )(h, ct, csq)

    logsm = pl.pallas_call(
        _tc3_body,
        grid=(_GRID,),
        in_specs=[_part_spec(F), _row_spec(K), _row_spec(F),
                  _full_spec((1, K))],
        out_specs=_row_spec(K),
        out_shape=jax.ShapeDtypeStruct((N, K), jnp.float32),
    )(aggc, h2p, dinv, b2r)

    return (logsm, q)


# speedup vs baseline: 1.2102x; 1.0296x over previous
import functools

import jax
import jax.numpy as jnp
from jax import lax
from jax.experimental import pallas as pl
from jax.experimental.pallas import tpu as pltpu
from jax.experimental.pallas import tpu_sc as plsc

N = 10000
E = 320000
F = 128
K = 16
NC = 2
NS = 16
NW = NC * NS
CB = 128
NCH = 80
EW = NCH * CB
E_PAD = NW * EW
N_ACC = N + 128
NBUF = 4
ZC = 80
NZCH = N // ZC


def _mesh():
    return plsc.VectorSubcoreMesh(core_axis_name="c", subcore_axis_name="s",
                                  num_cores=NC, num_subcores=NS)


def _zero_fill(zb, width):
    def body(i, carry):
        for k8 in range(width // 16):
            zb[i, pl.ds(k8 * 16, 16)] = jnp.zeros((16,), jnp.float32)
        return carry
    lax.fori_loop(0, ZC, body, 0)


def _zero_acc(zb, acc, sid):
    def body(k, carry):
        c = sid + k * NS

        @pl.when(c < NZCH)
        def _():
            pltpu.sync_copy(zb, acc.at[pl.ds(c * ZC, ZC)])
        return carry
    lax.fori_loop(0, (NZCH + NS - 1) // NS, body, 0)


def _write_out(acc, out_hbm, cid, width, h, sid):
    def body(k, carry):
        c = sid + k * NS

        @pl.when(c < NZCH)
        def _():
            pltpu.sync_copy(acc.at[pl.ds(c * ZC, ZC)],
                            out_hbm.at[cid, pl.ds(c * ZC, ZC),
                                       pl.ds(h * width, width)])
        return carry
    lax.fori_loop(0, (NZCH + NS - 1) // NS, body, 0)


def _sc_deg_body(dst_hbm, out_hbm, idx_v, ones_v, zb_v, acc, sem):
    cid = lax.axis_index("c")
    sid = lax.axis_index("s")
    wid = sid * NC + cid

    def fill_ones(i, carry):
        ones_v[i, :] = jnp.full((16,), 1.0, jnp.float32)
        return carry
    lax.fori_loop(0, CB, fill_ones, 0)
    _zero_fill(zb_v, K)
    _zero_acc(zb_v, acc, sid)
    plsc.subcore_barrier()

    pltpu.sync_copy(dst_hbm.at[wid], idx_v)

    def body(j, carry):
        pltpu.async_copy(ones_v, acc.at[idx_v.at[j]], sem, add=True)
        return carry
    lax.fori_loop(0, NCH, body, 0)

    def drain(j, carry):
        pltpu.make_async_copy(ones_v, acc.at[idx_v.at[j]], sem).wait()
        return carry
    lax.fori_loop(0, NCH, drain, 0)
    plsc.subcore_barrier()

    _write_out(acc, out_hbm, cid, K, 0, sid)


NBK = 8


def _sc_edge_k_body(rows_hbm, src_hbm, dst_hbm, out_hbm, idxs_v, idxd_v,
                    *refs):
    bufs = refs[:NBK]
    zb_v, acc = refs[NBK:NBK + 2]
    gsems = refs[NBK + 2:2 * NBK + 2]
    ssem = refs[2 * NBK + 2]
    cid = lax.axis_index("c")
    sid = lax.axis_index("s")
    wid = sid * NC + cid

    _zero_fill(zb_v, K)
    _zero_acc(zb_v, acc, sid)
    plsc.subcore_barrier()

    pltpu.sync_copy(src_hbm.at[wid], idxs_v)
    pltpu.sync_copy(dst_hbm.at[wid], idxd_v)

    def rnd(r, carry):
        j0 = r * NBK
        for b in range(NBK):
            pltpu.async_copy(rows_hbm.at[idxs_v.at[j0 + b]], bufs[b],
                             gsems[b])
        for b in range(NBK):
            pltpu.make_async_copy(rows_hbm.at[idxs_v.at[j0 + b]], bufs[b],
                                  gsems[b]).wait()
            pltpu.async_copy(bufs[b], acc.at[idxd_v.at[j0 + b]], ssem,
                             add=True)
        for b in range(NBK):
            pltpu.make_async_copy(bufs[b], acc.at[idxd_v.at[j0 + b]],
                                  ssem).wait()
        return carry
    lax.fori_loop(0, NCH // NBK, rnd, 0)
    plsc.subcore_barrier()

    _write_out(acc, out_hbm, cid, K, 0, sid)


NCH2 = E_PAD // (NS * CB)


def _sc_edge_split_body(rows_hbm, src_hbm, dst_hbm, out_hbm,
                        idxs_v, idxd_v, b0, b1, b2, b3, zb_v, acc,
                        s0, s1, s2, s3):
    bufs = (b0, b1, b2, b3)
    sems = (s0, s1, s2, s3)
    cid = lax.axis_index("c")
    sid = lax.axis_index("s")

    _zero_fill(zb_v, F // 2)
    _zero_acc(zb_v, acc, sid)

    pltpu.sync_copy(src_hbm.at[sid], idxs_v)
    pltpu.sync_copy(dst_hbm.at[sid], idxd_v)

    def xform(i, carry):
        for k8 in range(CB // 16):
            v = idxs_v[i, pl.ds(k8 * 16, 16)]
            idxs_v[i, pl.ds(k8 * 16, 16)] = v * 2 + cid
        return carry
    lax.fori_loop(0, NCH2, xform, 0)
    plsc.subcore_barrier()

    for b in range(NBUF):
        pltpu.async_copy(rows_hbm.at[idxs_v.at[b]], bufs[b], sems[b])

    def outer(o, carry):
        j0 = o * NBUF
        for b in range(NBUF):
            j = j0 + b
            pltpu.make_async_copy(rows_hbm.at[idxs_v.at[j]],
                                  bufs[b], sems[b]).wait()
            pltpu.sync_copy(bufs[b], acc.at[idxd_v.at[j]], add=True)

            @pl.when(j + NBUF < NCH2)
            def _():
                pltpu.async_copy(rows_hbm.at[idxs_v.at[j + NBUF]],
                                 bufs[b], sems[b])
        return carry
    lax.fori_loop(0, NCH2 // NBUF, outer, 0)
    plsc.subcore_barrier()

    def wout(k, carry):
        c = sid + k * NS

        @pl.when(c < NZCH)
        def _():
            pltpu.sync_copy(acc.at[pl.ds(c * ZC, ZC)],
                            out_hbm.at[pl.ds(c * ZC, ZC),
                                       pl.ds(cid * (F // 2), F // 2)])
        return carry
    lax.fori_loop(0, (NZCH + NS - 1) // NS, wout, 0)


def _make_edge_split_kernel():
    return pl.kernel(
        _sc_edge_split_body,
        out_type=jax.ShapeDtypeStruct((N, F), jnp.float32),
        mesh=_mesh(),
        compiler_params=pltpu.CompilerParams(use_tc_tiling_on_sc=False),
        scratch_types=(
            [pltpu.VMEM((NCH2, CB), jnp.int32),
             pltpu.VMEM((NCH2, CB), jnp.int32)]
            + [pltpu.VMEM((CB, F // 2), jnp.float32) for _ in range(NBUF)]
            + [pltpu.VMEM((ZC, F // 2), jnp.float32),
               pltpu.VMEM_SHARED((N_ACC, F // 2), jnp.float32)]
            + [pltpu.SemaphoreType.DMA for _ in range(NBUF)]
        ),
    )


def _make_deg_kernel():
    return pl.kernel(
        _sc_deg_body,
        out_type=jax.ShapeDtypeStruct((NC, N, F), jnp.float32),
        mesh=_mesh(),
        compiler_params=pltpu.CompilerParams(use_tc_tiling_on_sc=False),
        scratch_types=[
            pltpu.VMEM((NCH, CB), jnp.int32),
            pltpu.VMEM((CB, K), jnp.float32),
            pltpu.VMEM((ZC, K), jnp.float32),
            pltpu.VMEM_SHARED((N_ACC, K), jnp.float32),
            pltpu.SemaphoreType.DMA,
        ],
    )


def _make_edge_k_kernel():
    return pl.kernel(
        _sc_edge_k_body,
        out_type=jax.ShapeDtypeStruct((NC, N, F), jnp.float32),
        mesh=_mesh(),
        compiler_params=pltpu.CompilerParams(use_tc_tiling_on_sc=False),
        scratch_types=(
            [pltpu.VMEM((NCH, CB), jnp.int32),
             pltpu.VMEM((NCH, CB), jnp.int32)]
            + [pltpu.VMEM((CB, K), jnp.float32) for _ in range(NBK)]
            + [pltpu.VMEM((ZC, K), jnp.float32),
               pltpu.VMEM_SHARED((N_ACC, K), jnp.float32)]
            + [pltpu.SemaphoreType.DMA for _ in range(NBK)]
            + [pltpu.SemaphoreType.DMA]
        ),
    )


_BR = 1000


def _tc1a_body(x_ref, w1_ref, h1_ref):
    h1_ref[...] = jnp.dot(x_ref[...], w1_ref[...],
                          preferred_element_type=jnp.float32)


def _tc1b_body(h1_ref, degp_ref, h1p_ref, dinv_ref):
    deg = degp_ref[0, :, 0:1] + degp_ref[1, :, 0:1] + 1.0
    dinv = lax.rsqrt(deg)
    h1p_ref[...] = dinv * h1_ref[...]
    dinv_ref[...] = jnp.broadcast_to(dinv, (_BR, F))


def _tc2a_body(aggb_ref, h1p_ref, dinv_ref, b1_ref, w2_ref,
               h_ref, h2p_ref):
    dinv = dinv_ref[:, 0:1]
    agg = aggb_ref[...] + h1p_ref[...]
    h = jnp.maximum(dinv * agg + b1_ref[...], 0.0)
    h_ref[...] = h
    h2 = jnp.dot(h, w2_ref[...], preferred_element_type=jnp.float32)
    h2p_ref[...] = dinv * h2


def _tc2b_body(h_ref, ct_ref, csq_ref, q_ref):
    h = h_ref[...]
    hsq = jnp.sum(h * h, axis=1, keepdims=True)
    cross = jnp.dot(h, ct_ref[...], preferred_element_type=jnp.float32)
    dist = hsq - 2.0 * cross + csq_ref[...]
    qun = 1.0 / (1.0 + dist)
    q_ref[...] = qun / jnp.sum(qun, axis=1, keepdims=True)


def _tc3_body(aggc_ref, h2p_ref, dinv_ref, b2_ref, out_ref):
    aggc = aggc_ref[0, :, :K] + aggc_ref[1, :, :K]
    logits = dinv_ref[:, 0:1] * (aggc + h2p_ref[...])
    logits = logits + b2_ref[...]
    m = jnp.max(logits, axis=1, keepdims=True)
    s = logits - m
    out_ref[...] = s - jnp.log(jnp.sum(jnp.exp(s), axis=1, keepdims=True))


def _row_spec(width):
    return pl.BlockSpec((_BR, width), lambda i: (i, 0))


def _full_spec(shape):
    return pl.BlockSpec(shape, lambda i: tuple(0 for _ in shape))


def _part_spec(width):
    return pl.BlockSpec((NC, _BR, width), lambda i: (0, i, 0))


_GRID = N // _BR


def kernel(x, edge_index, W1, b1, W2, b2, cluster_centers):
    ei = edge_index.astype(jnp.int32)
    pad_ids = jnp.arange(E_PAD - E, dtype=jnp.int32)
    fill = jnp.stack([pad_ids % N, N + (pad_ids % 128)])
    ei = jnp.concatenate([ei, fill], axis=1)
    src3 = ei[0].reshape(NW, NCH, CB)
    dst3 = ei[1].reshape(NW, NCH, CB)
    src2 = ei[0].reshape(NS, NCH2, CB)
    dst2 = ei[1].reshape(NS, NCH2, CB)

    degp = _make_deg_kernel()(dst3)

    h1 = pl.pallas_call(
        _tc1a_body,
        grid=(_GRID,),
        in_specs=[_row_spec(F), _full_spec((F, F))],
        out_specs=_row_spec(F),
        out_shape=jax.ShapeDtypeStruct((N, F), jnp.float32),
    )(x, W1)

    h1p, dinv = pl.pallas_call(
        _tc1b_body,
        grid=(_GRID,),
        in_specs=[_row_spec(F), _part_spec(F)],
        out_specs=[_row_spec(F), _row_spec(F)],
        out_shape=[jax.ShapeDtypeStruct((N, F), jnp.float32),
                   jax.ShapeDtypeStruct((N, F), jnp.float32)],
    )(h1, degp)

    aggb = _make_edge_split_kernel()(
        h1p.reshape(2 * N, F // 2), src2, dst2)

    b1r = b1.reshape(1, F)
    b2r = b2.reshape(1, K)
    ct = cluster_centers.T
    csq = jnp.sum(cluster_centers * cluster_centers, axis=1).reshape(1, K)

    h, h2p = pl.pallas_call(
        _tc2a_body,
        grid=(_GRID,),
        in_specs=[_row_spec(F), _row_spec(F), _row_spec(F),
                  _full_spec((1, F)), _full_spec((F, K))],
        out_specs=[_row_spec(F), _row_spec(K)],
        out_shape=[jax.ShapeDtypeStruct((N, F), jnp.float32),
                   jax.ShapeDtypeStruct((N, K), jnp.float32)],
    )(aggb, h1p, dinv, b1r, W2)

    aggc = _make_edge_k_kernel()(h2p, src3, dst3)

    q = pl.pallas_call(
        _tc2b_body,
        grid=(_GRID,),
        in_specs=[_row_spec(F), _full_spec((F, K)), _full_spec((1, K))],
        out_specs=_row_spec(K),
        out_shape=jax.ShapeDtypeStruct((N, K), jnp.float32),
    )(h, ct, csq)

    logsm = pl.pallas_call(
        _tc3_body,
        grid=(_GRID,),
        in_specs=[_part_spec(F), _row_spec(K), _row_spec(F),
                  _full_spec((1, K))],
        out_specs=_row_spec(K),
        out_shape=jax.ShapeDtypeStruct((N, K), jnp.float32),
    )(aggc, h2p, dinv, b2r)

    return (logsm, q)
